# Initial kernel scaffold; baseline (speedup 1.0000x reference)
import jax
import jax.numpy as jnp
from jax.experimental import pallas as pl


def kernel(x, params):
    raise NotImplementedError("write your pallas kernel here")


# speedup vs baseline: 67.4219x; 67.4219x over previous
import jax
import jax.numpy as jnp
from jax.experimental import pallas as pl

_N = 8
_B = 64
_BB = 8
_G = _B // _BB
_RB = _BB * _N


def _lrelu(t):
    return jnp.where(t >= 0, t, 0.2 * t)


def _dot(a, b):
    return jnp.dot(a, b, preferred_element_type=jnp.float32)


def _gat(Xf, p, Mf, ea, loop_attr):
    cout = p['lin_l_W'].shape[1]
    xl = _dot(Xf, p['lin_l_W']) + p['lin_l_b']
    xr = _dot(Xf, p['lin_r_W']) + p['lin_r_b']
    xl3 = xl.reshape(_BB, _N, cout)
    xr3 = xr.reshape(_BB, _N, cout)
    we4 = p['lin_edge_W'].reshape(1, 1, 1, cout)
    att4 = p['att'].reshape(1, 1, 1, cout)
    T = xl3[:, None, :, :] + xr3[:, :, None, :] + ea[:, :, :, None] * we4
    alpha_t = jnp.sum(_lrelu(T) * att4, axis=-1)
    Ts = xl3 + xr3 + loop_attr[:, :, None] * p['lin_edge_W'].reshape(1, 1, cout)
    alpha_s = jnp.sum(_lrelu(Ts) * p['att'].reshape(1, 1, cout), axis=-1)
    neg = jnp.float32(-1e30)
    am_e = jnp.max(jnp.where(Mf > 0.5, alpha_t, neg), axis=-1)
    amax = jnp.maximum(am_e, alpha_s)
    ex = jnp.where(Mf > 0.5, jnp.exp(alpha_t - amax[:, :, None]), 0.0)
    ex_s = jnp.exp(alpha_s - amax)
    den = jnp.sum(ex, axis=-1) + ex_s
    inv = 1.0 / (den + 1e-16)
    a_t = ex * inv[:, :, None]
    a_s = ex_s * inv
    acc = a_s[:, :, None] * xl3
    for i in range(_N):
        acc = acc + a_t[:, :, i:i + 1] * xl3[:, i:i + 1, :]
    out3 = acc + p['bias'].reshape(1, 1, cout)
    return out3.reshape(_RB, cout)


def _make_body(treedef):
    def body(xp_ref, *refs):
        rec_ref, lat_ref = refs[-2], refs[-1]
        p = jax.tree_util.tree_unflatten(treedef, [r[...] for r in refs[:-2]])

        xp = xp_ref[...]
        h = jnp.maximum(_dot(xp, p['enc_W1']) + p['enc_b1'], 0.0)
        h = jnp.maximum(_dot(h, p['enc_W2']) + p['enc_b2'], 0.0)
        latent = _dot(h, p['enc_W3']) + p['enc_b3']

        lat3 = latent.reshape(_BB, _N, 3)
        mu = jnp.sum(lat3, axis=1, keepdims=True) * (1.0 / _N)
        ctr = lat3 - mu
        mu2 = jnp.sum(ctr, axis=1, keepdims=True) * (1.0 / _N)
        d = ctr - mu2
        std = jnp.sqrt(jnp.sum(d * d, axis=1, keepdims=True) * (1.0 / (_N - 1)))
        gl3 = ctr / (std + 1e-8)
        lat_ref[...] = gl3.reshape(_RB, 3)

        dif = gl3[:, :, None, :] - gl3[:, None, :, :]
        dist = jnp.sqrt(jnp.sum(dif * dif, axis=-1))
        radius = dist * 0.5
        dk2 = jnp.zeros((_BB, _N, _N, _N), jnp.float32)
        for c in range(3):
            glc = gl3[:, :, c]
            cen = 0.5 * (glc[:, :, None] + glc[:, None, :])
            t = glc[:, None, None, :] - cen[:, :, :, None]
            dk2 = dk2 + t * t
        dk = jnp.sqrt(dk2)
        ii = jax.lax.broadcasted_iota(jnp.int32, (_BB, _N, _N, _N), 1)
        jj = jax.lax.broadcasted_iota(jnp.int32, (_BB, _N, _N, _N), 2)
        kk = jax.lax.broadcasted_iota(jnp.int32, (_BB, _N, _N, _N), 3)
        excl = (kk == ii) | (kk == jj)
        blockedf = jnp.where((dk < radius[:, :, :, None]) & (~excl), 1.0, 0.0)
        okf = 1.0 - jnp.max(blockedf, axis=-1)
        Mf = jnp.where((dist >= 1e-10) & (okf > 0.5), 1.0, 0.0)

        difL = lat3[:, :, None, :] - lat3[:, None, :, :]
        ea = jnp.sqrt(jnp.sum(difL * difL, axis=-1))
        cnt = jnp.sum(Mf, axis=-1)
        loop_attr = jnp.sum(Mf * ea, axis=-1) / jnp.maximum(cnt, 1.0)

        nx = latent[:, 2:3]
        x1 = jnp.maximum(_gat(nx, p['g1'], Mf, ea, loop_attr), 0.0)
        x2 = jnp.maximum(_gat(x1, p['g2'], Mf, ea, loop_attr), 0.0)
        out3 = _gat(x2, p['g3'], Mf, ea, loop_attr)
        rec = out3 + 0.1 * (_dot(latent, p['skip_W']) + p['skip_b'])
        rec_ref[...] = rec

    return body


def kernel(x, params):
    B, N = x.shape
    idx = jnp.broadcast_to(jnp.arange(N, dtype=x.dtype), (B, N))
    z = jnp.zeros((B, N), x.dtype)
    xp = jnp.stack([z, idx, x], axis=-1)

    leaves, treedef = jax.tree_util.tree_flatten(params)
    leaves2d = [l.reshape(1, -1) if l.ndim == 1 else l for l in leaves]

    row_spec = pl.BlockSpec((_RB, 3), lambda i: (i, 0))
    param_specs = [pl.BlockSpec(l.shape, lambda i: (0, 0)) for l in leaves2d]
    rec_flat, lat_flat = pl.pallas_call(
        _make_body(treedef),
        grid=(_G,),
        in_specs=[row_spec] + param_specs,
        out_specs=[row_spec, row_spec],
        out_shape=[
            jax.ShapeDtypeStruct((_B * _N, 3), jnp.float32),
            jax.ShapeDtypeStruct((_B * _N, 3), jnp.float32),
        ],
    )(xp.reshape(_B * _N, 3), *leaves2d)

    return (xp, rec_flat.reshape(B, N, 3), lat_flat.reshape(B, N, 3))


# speedup vs baseline: 159.1695x; 2.3608x over previous
import jax
import jax.numpy as jnp
from jax.experimental import pallas as pl

_N = 8
_B = 64
_BB = 8
_G = _B // _BB
_RB = _BB * _N


def _lrelu(t):
    return jnp.where(t >= 0, t, 0.2 * t)


def _dot(a, b):
    return jnp.dot(a, b, preferred_element_type=jnp.float32)


def _gat(Xf, p, Mf, ea, loop_attr, bm):
    cout = p['lin_l_W'].shape[1]
    xl = _dot(Xf, p['lin_l_W']) + p['lin_l_b']
    xr = _dot(Xf, p['lin_r_W']) + p['lin_r_b']
    xl3 = xl.reshape(_BB, _N, cout)
    xr3 = xr.reshape(_BB, _N, cout)
    we4 = p['lin_edge_W'].reshape(1, 1, 1, cout)
    att4 = p['att'].reshape(1, 1, 1, cout)
    T = xl3[:, None, :, :] + xr3[:, :, None, :] + ea[:, :, :, None] * we4
    alpha_t = jnp.sum(_lrelu(T) * att4, axis=-1)
    Ts = xl3 + xr3 + loop_attr[:, :, None] * p['lin_edge_W'].reshape(1, 1, cout)
    alpha_s = jnp.sum(_lrelu(Ts) * p['att'].reshape(1, 1, cout), axis=-1)
    neg = jnp.float32(-1e30)
    am_e = jnp.max(jnp.where(Mf > 0.5, alpha_t, neg), axis=-1)
    amax = jnp.maximum(am_e, alpha_s)
    ex = jnp.where(Mf > 0.5, jnp.exp(alpha_t - amax[:, :, None]), 0.0)
    ex_s = jnp.exp(alpha_s - amax)
    den = jnp.sum(ex, axis=-1) + ex_s
    inv = 1.0 / (den + 1e-16)
    a_t = ex * inv[:, :, None]
    a_s = ex_s * inv
    jj2 = jax.lax.broadcasted_iota(jnp.int32, (_BB, _N, _N), 1)
    kk2 = jax.lax.broadcasted_iota(jnp.int32, (_BB, _N, _N), 2)
    a_comb = a_t + jnp.where(jj2 == kk2, a_s[:, :, None], 0.0)
    a2 = a_comb.reshape(_RB, _N)
    tiled = jnp.concatenate([a2] * _BB, axis=1)
    return _dot(tiled * bm, xl) + p['bias']


def _make_body(treedef):
    def body(xp_ref, *refs):
        rec_ref, lat_ref = refs[-2], refs[-1]
        p = jax.tree_util.tree_unflatten(treedef, [r[...] for r in refs[:-2]])

        xp = xp_ref[...]
        h = jnp.maximum(_dot(xp, p['enc_W1']) + p['enc_b1'], 0.0)
        h = jnp.maximum(_dot(h, p['enc_W2']) + p['enc_b2'], 0.0)
        latent = _dot(h, p['enc_W3']) + p['enc_b3']

        lat3 = latent.reshape(_BB, _N, 3)
        mu = jnp.sum(lat3, axis=1, keepdims=True) * (1.0 / _N)
        ctr = lat3 - mu
        mu2 = jnp.sum(ctr, axis=1, keepdims=True) * (1.0 / _N)
        d = ctr - mu2
        std = jnp.sqrt(jnp.sum(d * d, axis=1, keepdims=True) * (1.0 / (_N - 1)))
        gl3 = ctr / (std + 1e-8)
        lat_ref[...] = gl3.reshape(_RB, 3)

        dif = gl3[:, :, None, :] - gl3[:, None, :, :]
        dist = jnp.sqrt(jnp.sum(dif * dif, axis=-1))
        radius = dist * 0.5
        dk2 = jnp.zeros((_BB, _N, _N, _N), jnp.float32)
        for c in range(3):
            glc = gl3[:, :, c]
            cen = 0.5 * (glc[:, :, None] + glc[:, None, :])
            t = glc[:, None, None, :] - cen[:, :, :, None]
            dk2 = dk2 + t * t
        dk = jnp.sqrt(dk2)
        ii = jax.lax.broadcasted_iota(jnp.int32, (_BB, _N, _N, _N), 1)
        jj = jax.lax.broadcasted_iota(jnp.int32, (_BB, _N, _N, _N), 2)
        kk = jax.lax.broadcasted_iota(jnp.int32, (_BB, _N, _N, _N), 3)
        excl = (kk == ii) | (kk == jj)
        blockedf = jnp.where((dk < radius[:, :, :, None]) & (~excl), 1.0, 0.0)
        okf = 1.0 - jnp.max(blockedf, axis=-1)
        Mf = jnp.where((dist >= 1e-10) & (okf > 0.5), 1.0, 0.0)

        difL = lat3[:, :, None, :] - lat3[:, None, :, :]
        ea = jnp.sqrt(jnp.sum(difL * difL, axis=-1))
        cnt = jnp.sum(Mf, axis=-1)
        loop_attr = jnp.sum(Mf * ea, axis=-1) / jnp.maximum(cnt, 1.0)

        ri = jax.lax.broadcasted_iota(jnp.int32, (_RB, _RB), 0)
        ci = jax.lax.broadcasted_iota(jnp.int32, (_RB, _RB), 1)
        bm = jnp.where((ri // _N) == (ci // _N), 1.0, 0.0)

        nx = latent[:, 2:3]
        x1 = jnp.maximum(_gat(nx, p['g1'], Mf, ea, loop_attr, bm), 0.0)
        x2 = jnp.maximum(_gat(x1, p['g2'], Mf, ea, loop_attr, bm), 0.0)
        out3 = _gat(x2, p['g3'], Mf, ea, loop_attr, bm)
        rec = out3 + 0.1 * (_dot(latent, p['skip_W']) + p['skip_b'])
        rec_ref[...] = rec

    return body


def kernel(x, params):
    B, N = x.shape
    idx = jnp.broadcast_to(jnp.arange(N, dtype=x.dtype), (B, N))
    z = jnp.zeros((B, N), x.dtype)
    xp = jnp.stack([z, idx, x], axis=-1)

    leaves, treedef = jax.tree_util.tree_flatten(params)
    leaves2d = [l.reshape(1, -1) if l.ndim == 1 else l for l in leaves]

    row_spec = pl.BlockSpec((_RB, 3), lambda i: (i, 0))
    param_specs = [pl.BlockSpec(l.shape, lambda i: (0, 0)) for l in leaves2d]
    rec_flat, lat_flat = pl.pallas_call(
        _make_body(treedef),
        grid=(_G,),
        in_specs=[row_spec] + param_specs,
        out_specs=[row_spec, row_spec],
        out_shape=[
            jax.ShapeDtypeStruct((_B * _N, 3), jnp.float32),
            jax.ShapeDtypeStruct((_B * _N, 3), jnp.float32),
        ],
    )(xp.reshape(_B * _N, 3), *leaves2d)

    return (xp, rec_flat.reshape(B, N, 3), lat_flat.reshape(B, N, 3))


# speedup vs baseline: 256.6507x; 1.6124x over previous
import jax
import jax.numpy as jnp
from jax.experimental import pallas as pl

_N = 8
_B = 64
_BB = 8
_G = _B // _BB
_RB = _BB * _N


def _lrelu(t):
    return jnp.where(t >= 0, t, 0.2 * t)


def _dot(a, b):
    return jnp.dot(a, b, preferred_element_type=jnp.float32)


def _gat(Xf, p, Mf, ea, loop_attr, bm, eye3):
    cout = p['lin_l_W'].shape[1]
    xl = _dot(Xf, p['lin_l_W']) + p['lin_l_b']
    xr = _dot(Xf, p['lin_r_W']) + p['lin_r_b']
    xl3 = xl.reshape(_BB, _N, cout)
    xr3 = xr.reshape(_BB, _N, cout)
    we4 = p['lin_edge_W'].reshape(1, 1, 1, cout)
    att4 = p['att'].reshape(1, 1, 1, cout)
    T = xl3[:, None, :, :] + xr3[:, :, None, :] + ea[:, :, :, None] * we4
    alpha_t = jnp.sum(_lrelu(T) * att4, axis=-1)
    Ts = xl3 + xr3 + loop_attr[:, :, None] * p['lin_edge_W'].reshape(1, 1, cout)
    alpha_s = jnp.sum(_lrelu(Ts) * p['att'].reshape(1, 1, cout), axis=-1)
    neg = jnp.float32(-1e30)
    am_e = jnp.max(jnp.where(Mf > 0.5, alpha_t, neg), axis=-1)
    amax = jnp.maximum(am_e, alpha_s)
    ex = jnp.where(Mf > 0.5, jnp.exp(alpha_t - amax[:, :, None]), 0.0)
    ex_s = jnp.exp(alpha_s - amax)
    den = jnp.sum(ex, axis=-1) + ex_s
    inv = 1.0 / (den + 1e-16)
    a_t = ex * inv[:, :, None]
    a_s = ex_s * inv
    a_comb = a_t + eye3 * a_s[:, :, None]
    a2 = a_comb.reshape(_RB, _N)
    tiled = jnp.concatenate([a2] * _BB, axis=1)
    return _dot(tiled * bm, xl) + p['bias']


def _make_body(treedef):
    def body(xp_ref, notexcl_ref, eye3_ref, bm_ref, savg_ref, dmat_ref, *refs):
        rec_ref, lat_ref = refs[-2], refs[-1]
        p = jax.tree_util.tree_unflatten(treedef, [r[...] for r in refs[:-2]])
        notexcl = notexcl_ref[...]
        eye3 = eye3_ref[...]
        bm = bm_ref[...]
        savg = savg_ref[...]
        dmat = dmat_ref[...]

        xp = xp_ref[...]
        h = jnp.maximum(_dot(xp, p['enc_W1']) + p['enc_b1'], 0.0)
        h = jnp.maximum(_dot(h, p['enc_W2']) + p['enc_b2'], 0.0)
        latent = _dot(h, p['enc_W3']) + p['enc_b3']

        lat3 = latent.reshape(_BB, _N, 3)
        mu = jnp.sum(lat3, axis=1, keepdims=True) * (1.0 / _N)
        ctr = lat3 - mu
        mu2 = jnp.sum(ctr, axis=1, keepdims=True) * (1.0 / _N)
        d = ctr - mu2
        std = jnp.sqrt(jnp.sum(d * d, axis=1, keepdims=True) * (1.0 / (_N - 1)))
        gl3 = ctr / (std + 1e-8)
        lat_ref[...] = gl3.reshape(_RB, 3)

        dist2 = jnp.zeros((_BB, _N * _N), jnp.float32)
        ea2 = jnp.zeros((_BB, _N * _N), jnp.float32)
        dk2 = jnp.zeros((_BB, _N, _N * _N), jnp.float32)
        for c in range(3):
            glc = gl3[:, :, c]
            di = _dot(glc, dmat)
            dist2 = dist2 + di * di
            cen = _dot(glc, savg)
            t = glc[:, :, None] - cen[:, None, :]
            dk2 = dk2 + t * t
            lc = lat3[:, :, c]
            dl = _dot(lc, dmat)
            ea2 = ea2 + dl * dl
        dist = jnp.sqrt(dist2)
        radius = dist * 0.5
        dk = jnp.sqrt(dk2)
        blockedf = jnp.where(dk < radius[:, None, :], notexcl, 0.0)
        okf = 1.0 - jnp.max(blockedf, axis=1)
        mask_p = jnp.where((dist >= 1e-10) & (okf > 0.5), 1.0, 0.0)
        Mf = mask_p.reshape(_BB, _N, _N)

        ea = jnp.sqrt(ea2).reshape(_BB, _N, _N)
        cnt = jnp.sum(Mf, axis=-1)
        loop_attr = jnp.sum(Mf * ea, axis=-1) / jnp.maximum(cnt, 1.0)

        nx = latent[:, 2:3]
        x1 = jnp.maximum(_gat(nx, p['g1'], Mf, ea, loop_attr, bm, eye3), 0.0)
        x2 = jnp.maximum(_gat(x1, p['g2'], Mf, ea, loop_attr, bm, eye3), 0.0)
        out3 = _gat(x2, p['g3'], Mf, ea, loop_attr, bm, eye3)
        rec = out3 + 0.1 * (_dot(latent, p['skip_W']) + p['skip_b'])
        rec_ref[...] = rec

    return body


def kernel(x, params):
    B, N = x.shape
    idx = jnp.broadcast_to(jnp.arange(N, dtype=x.dtype), (B, N))
    z = jnp.zeros((B, N), x.dtype)
    xp = jnp.stack([z, idx, x], axis=-1)

    leaves, treedef = jax.tree_util.tree_flatten(params)
    leaves2d = [l.reshape(1, -1) if l.ndim == 1 else l for l in leaves]

    kk = jnp.arange(_N)
    ip = (jnp.arange(_N * _N) // _N)[None, :]
    jp = (jnp.arange(_N * _N) % _N)[None, :]
    k1 = kk[:, None]
    notexcl = jnp.where((k1 == ip) | (k1 == jp), 0.0, 1.0)[None]
    savg = 0.5 * ((k1 == ip).astype(jnp.float32) + (k1 == jp).astype(jnp.float32))
    dmat = (k1 == ip).astype(jnp.float32) - (k1 == jp).astype(jnp.float32)
    eye3 = jnp.eye(_N, dtype=jnp.float32)[None]
    r = jnp.arange(_RB)
    bm = ((r[:, None] // _N) == (r[None, :] // _N)).astype(jnp.float32)

    row_spec = pl.BlockSpec((_RB, 3), lambda i: (i, 0))
    param_specs = [pl.BlockSpec(l.shape, lambda i: (0, 0)) for l in leaves2d]
    const_specs = [
        pl.BlockSpec((1, _N, _N * _N), lambda i: (0, 0, 0)),
        pl.BlockSpec((1, _N, _N), lambda i: (0, 0, 0)),
        pl.BlockSpec((_RB, _RB), lambda i: (0, 0)),
        pl.BlockSpec((_N, _N * _N), lambda i: (0, 0)),
        pl.BlockSpec((_N, _N * _N), lambda i: (0, 0)),
    ]
    rec_flat, lat_flat = pl.pallas_call(
        _make_body(treedef),
        grid=(_G,),
        in_specs=[row_spec] + const_specs + param_specs,
        out_specs=[row_spec, row_spec],
        out_shape=[
            jax.ShapeDtypeStruct((_B * _N, 3), jnp.float32),
            jax.ShapeDtypeStruct((_B * _N, 3), jnp.float32),
        ],
    )(xp.reshape(_B * _N, 3), notexcl, eye3, bm, savg, dmat, *leaves2d)

    return (xp, rec_flat.reshape(B, N, 3), lat_flat.reshape(B, N, 3))


# speedup vs baseline: 358.5965x; 1.3972x over previous
import jax
import jax.numpy as jnp
from jax.experimental import pallas as pl

_N = 8
_B = 64
_BB = 64
_G = _B // _BB
_RB = _BB * _N


def _lrelu(t):
    return jnp.where(t >= 0, t, 0.2 * t)


def _dot(a, b):
    return jnp.dot(a, b, preferred_element_type=jnp.float32)


def _gat(Xf, p, Mf, ea, loop_attr, bm, eye3):
    cout = p['lin_l_W'].shape[1]
    xl = _dot(Xf, p['lin_l_W']) + p['lin_l_b']
    xr = _dot(Xf, p['lin_r_W']) + p['lin_r_b']
    xl3 = xl.reshape(_BB, _N, cout)
    xr3 = xr.reshape(_BB, _N, cout)
    we4 = p['lin_edge_W'].reshape(1, 1, 1, cout)
    att4 = p['att'].reshape(1, 1, 1, cout)
    T = xl3[:, None, :, :] + xr3[:, :, None, :] + ea[:, :, :, None] * we4
    alpha_t = jnp.sum(_lrelu(T) * att4, axis=-1)
    Ts = xl3 + xr3 + loop_attr[:, :, None] * p['lin_edge_W'].reshape(1, 1, cout)
    alpha_s = jnp.sum(_lrelu(Ts) * p['att'].reshape(1, 1, cout), axis=-1)
    neg = jnp.float32(-1e30)
    am_e = jnp.max(jnp.where(Mf > 0.5, alpha_t, neg), axis=-1)
    amax = jnp.maximum(am_e, alpha_s)
    ex = jnp.where(Mf > 0.5, jnp.exp(alpha_t - amax[:, :, None]), 0.0)
    ex_s = jnp.exp(alpha_s - amax)
    den = jnp.sum(ex, axis=-1) + ex_s
    inv = 1.0 / (den + 1e-16)
    a_t = ex * inv[:, :, None]
    a_s = ex_s * inv
    a_comb = a_t + eye3 * a_s[:, :, None]
    a2 = a_comb.reshape(_RB, _N)
    tiled = jnp.concatenate([a2] * _BB, axis=1)
    return _dot(tiled * bm, xl) + p['bias']


def _make_body(treedef):
    def body(xp_ref, notexcl_ref, eye3_ref, bm_ref, savg_ref, dmat_ref, *refs):
        rec_ref, lat_ref = refs[-2], refs[-1]
        p = jax.tree_util.tree_unflatten(treedef, [r[...] for r in refs[:-2]])
        notexcl = notexcl_ref[...]
        eye3 = eye3_ref[...]
        bm = bm_ref[...]
        savg = savg_ref[...]
        dmat = dmat_ref[...]

        xp = xp_ref[...]
        h = jnp.maximum(_dot(xp, p['enc_W1']) + p['enc_b1'], 0.0)
        h = jnp.maximum(_dot(h, p['enc_W2']) + p['enc_b2'], 0.0)
        latent = _dot(h, p['enc_W3']) + p['enc_b3']

        lat3 = latent.reshape(_BB, _N, 3)
        mu = jnp.sum(lat3, axis=1, keepdims=True) * (1.0 / _N)
        ctr = lat3 - mu
        mu2 = jnp.sum(ctr, axis=1, keepdims=True) * (1.0 / _N)
        d = ctr - mu2
        std = jnp.sqrt(jnp.sum(d * d, axis=1, keepdims=True) * (1.0 / (_N - 1)))
        gl3 = ctr / (std + 1e-8)
        lat_ref[...] = gl3.reshape(_RB, 3)

        dist2 = jnp.zeros((_BB, _N * _N), jnp.float32)
        ea2 = jnp.zeros((_BB, _N * _N), jnp.float32)
        dk2 = jnp.zeros((_BB, _N, _N * _N), jnp.float32)
        for c in range(3):
            glc = gl3[:, :, c]
            di = _dot(glc, dmat)
            dist2 = dist2 + di * di
            cen = _dot(glc, savg)
            t = glc[:, :, None] - cen[:, None, :]
            dk2 = dk2 + t * t
            lc = lat3[:, :, c]
            dl = _dot(lc, dmat)
            ea2 = ea2 + dl * dl
        dist = jnp.sqrt(dist2)
        radius = dist * 0.5
        dk = jnp.sqrt(dk2)
        blockedf = jnp.where(dk < radius[:, None, :], notexcl, 0.0)
        okf = 1.0 - jnp.max(blockedf, axis=1)
        mask_p = jnp.where((dist >= 1e-10) & (okf > 0.5), 1.0, 0.0)
        Mf = mask_p.reshape(_BB, _N, _N)

        ea = jnp.sqrt(ea2).reshape(_BB, _N, _N)
        cnt = jnp.sum(Mf, axis=-1)
        loop_attr = jnp.sum(Mf * ea, axis=-1) / jnp.maximum(cnt, 1.0)

        nx = latent[:, 2:3]
        x1 = jnp.maximum(_gat(nx, p['g1'], Mf, ea, loop_attr, bm, eye3), 0.0)
        x2 = jnp.maximum(_gat(x1, p['g2'], Mf, ea, loop_attr, bm, eye3), 0.0)
        out3 = _gat(x2, p['g3'], Mf, ea, loop_attr, bm, eye3)
        rec = out3 + 0.1 * (_dot(latent, p['skip_W']) + p['skip_b'])
        rec_ref[...] = rec

    return body


def kernel(x, params):
    B, N = x.shape
    idx = jnp.broadcast_to(jnp.arange(N, dtype=x.dtype), (B, N))
    z = jnp.zeros((B, N), x.dtype)
    xp = jnp.stack([z, idx, x], axis=-1)

    leaves, treedef = jax.tree_util.tree_flatten(params)
    leaves2d = [l.reshape(1, -1) if l.ndim == 1 else l for l in leaves]

    kk = jnp.arange(_N)
    ip = (jnp.arange(_N * _N) // _N)[None, :]
    jp = (jnp.arange(_N * _N) % _N)[None, :]
    k1 = kk[:, None]
    notexcl = jnp.where((k1 == ip) | (k1 == jp), 0.0, 1.0)[None]
    savg = 0.5 * ((k1 == ip).astype(jnp.float32) + (k1 == jp).astype(jnp.float32))
    dmat = (k1 == ip).astype(jnp.float32) - (k1 == jp).astype(jnp.float32)
    eye3 = jnp.eye(_N, dtype=jnp.float32)[None]
    r = jnp.arange(_RB)
    bm = ((r[:, None] // _N) == (r[None, :] // _N)).astype(jnp.float32)

    row_spec = pl.BlockSpec((_RB, 3), lambda i: (i, 0))
    param_specs = [pl.BlockSpec(l.shape, lambda i: (0, 0)) for l in leaves2d]
    const_specs = [
        pl.BlockSpec((1, _N, _N * _N), lambda i: (0, 0, 0)),
        pl.BlockSpec((1, _N, _N), lambda i: (0, 0, 0)),
        pl.BlockSpec((_RB, _RB), lambda i: (0, 0)),
        pl.BlockSpec((_N, _N * _N), lambda i: (0, 0)),
        pl.BlockSpec((_N, _N * _N), lambda i: (0, 0)),
    ]
    rec_flat, lat_flat = pl.pallas_call(
        _make_body(treedef),
        grid=(_G,),
        in_specs=[row_spec] + const_specs + param_specs,
        out_specs=[row_spec, row_spec],
        out_shape=[
            jax.ShapeDtypeStruct((_B * _N, 3), jnp.float32),
            jax.ShapeDtypeStruct((_B * _N, 3), jnp.float32),
        ],
    )(xp.reshape(_B * _N, 3), notexcl, eye3, bm, savg, dmat, *leaves2d)

    return (xp, rec_flat.reshape(B, N, 3), lat_flat.reshape(B, N, 3))


# speedup vs baseline: 394.4844x; 1.1001x over previous
import jax
import jax.numpy as jnp
import numpy as np
from jax.experimental import pallas as pl

_N = 8
_B = 64
_BB = 64
_G = _B // _BB
_RB = _BB * _N


def _lrelu(t):
    return jnp.where(t >= 0, t, 0.2 * t)


def _dot(a, b):
    return jnp.dot(a, b, preferred_element_type=jnp.float32)


def _gat(Xf, p, Mf, ea, loop_attr, bm, eye3):
    cout = p['lin_l_W'].shape[1]
    xl = _dot(Xf, p['lin_l_W']) + p['lin_l_b']
    xr = _dot(Xf, p['lin_r_W']) + p['lin_r_b']
    xl3 = xl.reshape(_BB, _N, cout)
    xr3 = xr.reshape(_BB, _N, cout)
    we4 = p['lin_edge_W'].reshape(1, 1, 1, cout)
    att4 = p['att'].reshape(1, 1, 1, cout)
    T = xl3[:, None, :, :] + xr3[:, :, None, :] + ea[:, :, :, None] * we4
    alpha_t = jnp.sum(_lrelu(T) * att4, axis=-1)
    Ts = xl3 + xr3 + loop_attr[:, :, None] * p['lin_edge_W'].reshape(1, 1, cout)
    alpha_s = jnp.sum(_lrelu(Ts) * p['att'].reshape(1, 1, cout), axis=-1)
    neg = jnp.float32(-1e30)
    am_e = jnp.max(jnp.where(Mf > 0.5, alpha_t, neg), axis=-1)
    amax = jnp.maximum(am_e, alpha_s)
    ex = jnp.where(Mf > 0.5, jnp.exp(alpha_t - amax[:, :, None]), 0.0)
    ex_s = jnp.exp(alpha_s - amax)
    den = jnp.sum(ex, axis=-1) + ex_s
    inv = 1.0 / (den + 1e-16)
    a_t = ex * inv[:, :, None]
    a_s = ex_s * inv
    a_comb = a_t + eye3 * a_s[:, :, None]
    a2 = a_comb.reshape(_RB, _N)
    tiled = jnp.concatenate([a2] * _BB, axis=1)
    return _dot(tiled * bm, xl) + p['bias']


def _make_body(treedef):
    def body(x_ref, prefix_ref, notexcl_ref, eye3_ref, bm_ref, savg_ref, dmat_ref, *refs):
        xp_ref, rec_ref, lat_ref = refs[-3], refs[-2], refs[-1]
        p = jax.tree_util.tree_unflatten(treedef, [r[...] for r in refs[:-3]])
        notexcl = notexcl_ref[...]
        eye3 = eye3_ref[...]
        bm = bm_ref[...]
        savg = savg_ref[...]
        dmat = dmat_ref[...]

        xp = jnp.concatenate([prefix_ref[...], x_ref[...]], axis=1)
        xp_ref[...] = xp
        h = jnp.maximum(_dot(xp, p['enc_W1']) + p['enc_b1'], 0.0)
        h = jnp.maximum(_dot(h, p['enc_W2']) + p['enc_b2'], 0.0)
        latent = _dot(h, p['enc_W3']) + p['enc_b3']

        lat3 = latent.reshape(_BB, _N, 3)
        mu = jnp.sum(lat3, axis=1, keepdims=True) * (1.0 / _N)
        ctr = lat3 - mu
        mu2 = jnp.sum(ctr, axis=1, keepdims=True) * (1.0 / _N)
        d = ctr - mu2
        std = jnp.sqrt(jnp.sum(d * d, axis=1, keepdims=True) * (1.0 / (_N - 1)))
        gl3 = ctr / (std + 1e-8)
        lat_ref[...] = gl3.reshape(_RB, 3)

        dist2 = jnp.zeros((_BB, _N * _N), jnp.float32)
        ea2 = jnp.zeros((_BB, _N * _N), jnp.float32)
        dk2 = jnp.zeros((_BB, _N, _N * _N), jnp.float32)
        for c in range(3):
            glc = gl3[:, :, c]
            di = _dot(glc, dmat)
            dist2 = dist2 + di * di
            cen = _dot(glc, savg)
            t = glc[:, :, None] - cen[:, None, :]
            dk2 = dk2 + t * t
            lc = lat3[:, :, c]
            dl = _dot(lc, dmat)
            ea2 = ea2 + dl * dl
        dist = jnp.sqrt(dist2)
        radius = dist * 0.5
        dk = jnp.sqrt(dk2)
        blockedf = jnp.where(dk < radius[:, None, :], notexcl, 0.0)
        okf = 1.0 - jnp.max(blockedf, axis=1)
        mask_p = jnp.where((dist >= 1e-10) & (okf > 0.5), 1.0, 0.0)
        Mf = mask_p.reshape(_BB, _N, _N)

        ea = jnp.sqrt(ea2).reshape(_BB, _N, _N)
        cnt = jnp.sum(Mf, axis=-1)
        loop_attr = jnp.sum(Mf * ea, axis=-1) / jnp.maximum(cnt, 1.0)

        nx = latent[:, 2:3]
        x1 = jnp.maximum(_gat(nx, p['g1'], Mf, ea, loop_attr, bm, eye3), 0.0)
        x2 = jnp.maximum(_gat(x1, p['g2'], Mf, ea, loop_attr, bm, eye3), 0.0)
        out3 = _gat(x2, p['g3'], Mf, ea, loop_attr, bm, eye3)
        rec = out3 + 0.1 * (_dot(latent, p['skip_W']) + p['skip_b'])
        rec_ref[...] = rec

    return body


_kk = np.arange(_N)
_ipn = (np.arange(_N * _N) // _N)[None, :]
_jpn = (np.arange(_N * _N) % _N)[None, :]
_k1 = _kk[:, None]
_NOTEXCL = np.where((_k1 == _ipn) | (_k1 == _jpn), 0.0, 1.0).astype(np.float32)[None]
_SAVG = (0.5 * ((_k1 == _ipn) + (_k1 == _jpn))).astype(np.float32)
_DMAT = ((_k1 == _ipn).astype(np.float32) - (_k1 == _jpn).astype(np.float32))
_EYE3 = np.eye(_N, dtype=np.float32)[None]
_r = np.arange(_RB)
_BM = ((_r[:, None] // _N) == (_r[None, :] // _N)).astype(np.float32)
_PREFIX = np.stack(
    [np.zeros(_RB, np.float32), np.tile(np.arange(_N, dtype=np.float32), _B)],
    axis=1)


def kernel(x, params):
    B, N = x.shape
    leaves, treedef = jax.tree_util.tree_flatten(params)
    leaves2d = [l.reshape(1, -1) if l.ndim == 1 else l for l in leaves]

    row_spec = pl.BlockSpec((_RB, 3), lambda i: (i, 0))
    param_specs = [pl.BlockSpec(l.shape, lambda i: (0, 0)) for l in leaves2d]
    const_specs = [
        pl.BlockSpec((_RB, 2), lambda i: (0, 0)),
        pl.BlockSpec((1, _N, _N * _N), lambda i: (0, 0, 0)),
        pl.BlockSpec((1, _N, _N), lambda i: (0, 0, 0)),
        pl.BlockSpec((_RB, _RB), lambda i: (0, 0)),
        pl.BlockSpec((_N, _N * _N), lambda i: (0, 0)),
        pl.BlockSpec((_N, _N * _N), lambda i: (0, 0)),
    ]
    xp_flat, rec_flat, lat_flat = pl.pallas_call(
        _make_body(treedef),
        grid=(_G,),
        in_specs=[pl.BlockSpec((_RB, 1), lambda i: (i, 0))] + const_specs + param_specs,
        out_specs=[row_spec, row_spec, row_spec],
        out_shape=[
            jax.ShapeDtypeStruct((_B * _N, 3), jnp.float32),
            jax.ShapeDtypeStruct((_B * _N, 3), jnp.float32),
            jax.ShapeDtypeStruct((_B * _N, 3), jnp.float32),
        ],
    )(x.reshape(_B * _N, 1), jnp.asarray(_PREFIX), jnp.asarray(_NOTEXCL),
      jnp.asarray(_EYE3), jnp.asarray(_BM), jnp.asarray(_SAVG),
      jnp.asarray(_DMAT), *leaves2d)

    return (xp_flat.reshape(B, N, 3), rec_flat.reshape(B, N, 3),
            lat_flat.reshape(B, N, 3))


# speedup vs baseline: 403.3262x; 1.0224x over previous
import jax
import jax.numpy as jnp
import numpy as np
from jax.experimental import pallas as pl

_N = 8
_B = 64
_RB = _B * _N
_P = _N * _N


def _lrelu(t):
    return jnp.where(t >= 0, t, 0.2 * t)


def _dot(a, b):
    return jnp.dot(a, b, preferred_element_type=jnp.float32)


def _gat(Xf, g, mask_r, ea3, ea_r, loop_col, bm, tile8):
    cout = g['lW'].shape[1]
    xl = _dot(Xf, g['lW']) + g['lb']
    xr = _dot(Xf, g['rW']) + g['rb']
    xl3 = xl.reshape(_B, _N, cout)
    xr3 = xr.reshape(_B, _N, cout)
    we4 = g['eW'].reshape(1, 1, 1, cout)
    att4 = g['att'].reshape(1, 1, 1, cout)
    T = xl3[:, None, :, :] + xr3[:, :, None, :] + ea3[:, :, :, None] * we4
    alpha_r = jnp.sum(_lrelu(T) * att4, axis=-1).reshape(_RB, _N)
    Ts = xl + xr + loop_col * g['eW']
    alpha_s = _dot(_lrelu(Ts), g['attc'])
    neg = jnp.float32(-1e30)
    am = jnp.max(jnp.where(mask_r > 0.5, alpha_r, neg), axis=1, keepdims=True)
    amax = jnp.maximum(am, alpha_s)
    ex_r = jnp.where(mask_r > 0.5, jnp.exp(alpha_r - amax), 0.0)
    ex_s = jnp.exp(alpha_s - amax)
    den = jnp.sum(ex_r, axis=1, keepdims=True) + ex_s
    inv = 1.0 / (den + 1e-16)
    agg = _dot(_dot(ex_r, tile8) * bm, xl)
    return (agg + ex_s * xl) * inv + g['bias']


def _body(x_ref, prefix_ref, notexcl_ref, bm_ref, savg_ref, dmat_ref,
          tile8_ref, enc_W1, enc_b1, enc_W2, enc_b2, enc_W3, enc_b3,
          skip_W, skip_b, *grefs):
    xp_ref, rec_ref, lat_ref = grefs[-3], grefs[-2], grefs[-1]
    gs = []
    for k in range(3):
        r = grefs[k * 8:(k + 1) * 8]
        gs.append({'lW': r[0][...], 'lb': r[1][...], 'rW': r[2][...],
                   'rb': r[3][...], 'eW': r[4][...], 'att': r[5][...],
                   'attc': r[6][...], 'bias': r[7][...]})
    notexcl = notexcl_ref[...]
    bm = bm_ref[...]
    savg = savg_ref[...]
    dmat = dmat_ref[...]
    tile8 = tile8_ref[...]

    xp = jnp.concatenate([prefix_ref[...], x_ref[...]], axis=1)
    xp_ref[...] = xp
    h = jnp.maximum(_dot(xp, enc_W1[...]) + enc_b1[...], 0.0)
    h = jnp.maximum(_dot(h, enc_W2[...]) + enc_b2[...], 0.0)
    latent = _dot(h, enc_W3[...]) + enc_b3[...]

    lat3 = latent.reshape(_B, _N, 3)
    mu = jnp.sum(lat3, axis=1, keepdims=True) * (1.0 / _N)
    ctr = lat3 - mu
    mu2 = jnp.sum(ctr, axis=1, keepdims=True) * (1.0 / _N)
    d = ctr - mu2
    std = jnp.sqrt(jnp.sum(d * d, axis=1, keepdims=True) * (1.0 / (_N - 1)))
    gl3 = ctr / (std + 1e-8)
    lat_ref[...] = gl3.reshape(_RB, 3)

    dist2 = jnp.zeros((_B, _P), jnp.float32)
    ea2 = jnp.zeros((_B, _P), jnp.float32)
    dk2 = jnp.zeros((_B, _N, _P), jnp.float32)
    for c in range(3):
        glc = gl3[:, :, c]
        di = _dot(glc, dmat)
        dist2 = dist2 + di * di
        cen = _dot(glc, savg)
        t = glc[:, :, None] - cen[:, None, :]
        dk2 = dk2 + t * t
        lc = lat3[:, :, c]
        dl = _dot(lc, dmat)
        ea2 = ea2 + dl * dl
    dist = jnp.sqrt(dist2)
    radius = dist * 0.5
    dk = jnp.sqrt(dk2)
    blockedf = jnp.where(dk < radius[:, None, :], notexcl, 0.0)
    okf = 1.0 - jnp.max(blockedf, axis=1)
    dist3 = dist.reshape(_B, _N, _N)
    okf3 = okf.reshape(_B, _N, _N)
    mask3 = jnp.where((dist3 >= 1e-10) & (okf3 > 0.5), 1.0, 0.0)
    mask_r = mask3.reshape(_RB, _N)

    ea3 = jnp.sqrt(ea2).reshape(_B, _N, _N)
    ea_r = jnp.sqrt(ea2.reshape(_B, _N, _N)).reshape(_RB, _N)
    mea = mask_r * ea_r
    cnt = jnp.sum(mask_r, axis=1, keepdims=True)
    loop_col = jnp.sum(mea, axis=1, keepdims=True) / jnp.maximum(cnt, 1.0)

    nx = latent[:, 2:3]
    x1 = jnp.maximum(_gat(nx, gs[0], mask_r, ea3, ea_r, loop_col, bm, tile8), 0.0)
    x2 = jnp.maximum(_gat(x1, gs[1], mask_r, ea3, ea_r, loop_col, bm, tile8), 0.0)
    out3 = _gat(x2, gs[2], mask_r, ea3, ea_r, loop_col, bm, tile8)
    rec = out3 + 0.1 * (_dot(latent, skip_W[...]) + skip_b[...])
    rec_ref[...] = rec


_kk = np.arange(_N)
_ipn = (np.arange(_P) // _N)[None, :]
_jpn = (np.arange(_P) % _N)[None, :]
_k1 = _kk[:, None]
_NOTEXCL = np.where((_k1 == _ipn) | (_k1 == _jpn), 0.0, 1.0).astype(np.float32)[None]
_SAVG = (0.5 * ((_k1 == _ipn) + (_k1 == _jpn))).astype(np.float32)
_DMAT = ((_k1 == _ipn).astype(np.float32) - (_k1 == _jpn).astype(np.float32))
_r = np.arange(_RB)
_BM = ((_r[:, None] // _N) == (_r[None, :] // _N)).astype(np.float32)
_TILE8 = (_kk[:, None] == (_r[None, :] % _N)).astype(np.float32)
_PREFIX = np.stack(
    [np.zeros(_RB, np.float32), np.tile(np.arange(_N, dtype=np.float32), _B)],
    axis=1)


def _full(shape):
    nd = len(shape)
    return pl.BlockSpec(shape, lambda i, _nd=nd: (0,) * _nd)


def kernel(x, params):
    B, N = x.shape
    p = params
    plist = [p['enc_W1'], p['enc_b1'].reshape(1, -1),
             p['enc_W2'], p['enc_b2'].reshape(1, -1),
             p['enc_W3'], p['enc_b3'].reshape(1, -1),
             p['skip_W'], p['skip_b'].reshape(1, -1)]
    for gk in ('g1', 'g2', 'g3'):
        g = p[gk]
        plist += [g['lin_l_W'], g['lin_l_b'].reshape(1, -1),
                  g['lin_r_W'], g['lin_r_b'].reshape(1, -1),
                  g['lin_edge_W'],
                  g['att'].reshape(1, -1), g['att'].reshape(-1, 1),
                  g['bias'].reshape(1, -1)]

    row_spec = pl.BlockSpec((_RB, 3), lambda i: (i, 0))
    const_in = [jnp.asarray(c) for c in
                (_PREFIX, _NOTEXCL, _BM, _SAVG, _DMAT, _TILE8)]
    xp_flat, rec_flat, lat_flat = pl.pallas_call(
        _body,
        grid=(1,),
        in_specs=([pl.BlockSpec((_RB, 1), lambda i: (i, 0))]
                  + [_full(c.shape) for c in const_in]
                  + [_full(l.shape) for l in plist]),
        out_specs=[row_spec, row_spec, row_spec],
        out_shape=[
            jax.ShapeDtypeStruct((_RB, 3), jnp.float32),
            jax.ShapeDtypeStruct((_RB, 3), jnp.float32),
            jax.ShapeDtypeStruct((_RB, 3), jnp.float32),
        ],
    )(x.reshape(_RB, 1), *const_in, *plist)

    return (xp_flat.reshape(B, N, 3), rec_flat.reshape(B, N, 3),
            lat_flat.reshape(B, N, 3))


# speedup vs baseline: 455.8218x; 1.1302x over previous
import jax
import jax.numpy as jnp
import numpy as np
from jax.experimental import pallas as pl

_N = 8
_B = 64
_RB = _B * _N
_P = _N * _N


def _lrelu(t):
    return jnp.where(t >= 0, t, 0.2 * t)


def _dot(a, b):
    return jnp.dot(a, b, preferred_element_type=jnp.float32)


def _gat(Xf, g, mask_r, ea_r, loop_col, bm, tile8, expi, bm8, e8v):
    cout = g['lW'].shape[1]
    xl = _dot(Xf, g['lW']) + g['lb']
    xr = _dot(Xf, g['rW']) + g['rb']
    xl3 = xl.reshape(_B, _N, cout)
    xl_wide = jnp.concatenate(
        [_dot(e8v, xl3[:, i, :]) for i in range(_N)], axis=1)
    xr_wide = jnp.concatenate([xr] * _N, axis=1)
    ea_wide = _dot(ea_r, expi)
    we_wide = jnp.concatenate([g['eW']] * _N, axis=1)
    Tw = _lrelu(xl_wide + xr_wide + ea_wide * we_wide)
    attblk = jnp.concatenate([g['attc']] * _N, axis=0) * bm8
    alpha_r = _dot(Tw, attblk)
    Ts = xl + xr + loop_col * g['eW']
    alpha_s = _dot(_lrelu(Ts), g['attc'])
    neg = jnp.float32(-1e30)
    am = jnp.max(jnp.where(mask_r > 0.5, alpha_r, neg), axis=1, keepdims=True)
    amax = jnp.maximum(am, alpha_s)
    ex_r = jnp.where(mask_r > 0.5, jnp.exp(alpha_r - amax), 0.0)
    ex_s = jnp.exp(alpha_s - amax)
    den = jnp.sum(ex_r, axis=1, keepdims=True) + ex_s
    inv = 1.0 / (den + 1e-16)
    agg = _dot(_dot(ex_r, tile8) * bm, xl)
    return (agg + ex_s * xl) * inv + g['bias']


def _body(x_ref, prefix_ref, notexcl_ref, bm_ref, savg_ref, dmat_ref,
          tile8_ref, expi64_ref, bm8_64_ref, expi3_ref, bm8_3_ref, e8v_ref,
          enc_W1, enc_b1, enc_W2, enc_b2, enc_W3, enc_b3,
          skip_W, skip_b, *grefs):
    xp_ref, rec_ref, lat_ref = grefs[-3], grefs[-2], grefs[-1]
    gs = []
    for k in range(3):
        r = grefs[k * 8:(k + 1) * 8]
        gs.append({'lW': r[0][...], 'lb': r[1][...], 'rW': r[2][...],
                   'rb': r[3][...], 'eW': r[4][...], 'att': r[5][...],
                   'attc': r[6][...], 'bias': r[7][...]})
    notexcl = notexcl_ref[...]
    bm = bm_ref[...]
    savg = savg_ref[...]
    dmat = dmat_ref[...]
    tile8 = tile8_ref[...]
    expi64 = expi64_ref[...]
    bm8_64 = bm8_64_ref[...]
    expi3 = expi3_ref[...]
    bm8_3 = bm8_3_ref[...]
    e8v = e8v_ref[...]

    xp = jnp.concatenate([prefix_ref[...], x_ref[...]], axis=1)
    xp_ref[...] = xp
    h = jnp.maximum(_dot(xp, enc_W1[...]) + enc_b1[...], 0.0)
    h = jnp.maximum(_dot(h, enc_W2[...]) + enc_b2[...], 0.0)
    latent = _dot(h, enc_W3[...]) + enc_b3[...]

    lat3 = latent.reshape(_B, _N, 3)
    mu = jnp.sum(lat3, axis=1, keepdims=True) * (1.0 / _N)
    ctr = lat3 - mu
    mu2 = jnp.sum(ctr, axis=1, keepdims=True) * (1.0 / _N)
    d = ctr - mu2
    std = jnp.sqrt(jnp.sum(d * d, axis=1, keepdims=True) * (1.0 / (_N - 1)))
    gl3 = ctr / (std + 1e-8)
    lat_ref[...] = gl3.reshape(_RB, 3)

    dist2 = jnp.zeros((_B, _P), jnp.float32)
    ea2 = jnp.zeros((_B, _P), jnp.float32)
    dk2 = jnp.zeros((_B, _N, _P), jnp.float32)
    for c in range(3):
        glc = gl3[:, :, c]
        di = _dot(glc, dmat)
        dist2 = dist2 + di * di
        cen = _dot(glc, savg)
        t = glc[:, :, None] - cen[:, None, :]
        dk2 = dk2 + t * t
        lc = lat3[:, :, c]
        dl = _dot(lc, dmat)
        ea2 = ea2 + dl * dl
    radius2 = dist2 * 0.25
    blockedf = jnp.where(dk2 < radius2[:, None, :], notexcl, 0.0)
    okf = 1.0 - jnp.max(blockedf, axis=1)
    dist3 = dist2.reshape(_B, _N, _N)
    okf3 = okf.reshape(_B, _N, _N)
    mask3 = jnp.where((dist3 >= 1e-20) & (okf3 > 0.5), 1.0, 0.0)
    mask_r = mask3.reshape(_RB, _N)

    ea_r = jnp.sqrt(ea2.reshape(_B, _N, _N)).reshape(_RB, _N)
    mea = mask_r * ea_r
    cnt = jnp.sum(mask_r, axis=1, keepdims=True)
    loop_col = jnp.sum(mea, axis=1, keepdims=True) / jnp.maximum(cnt, 1.0)

    nx = latent[:, 2:3]
    x1 = jnp.maximum(
        _gat(nx, gs[0], mask_r, ea_r, loop_col, bm, tile8, expi64, bm8_64, e8v), 0.0)
    x2 = jnp.maximum(
        _gat(x1, gs[1], mask_r, ea_r, loop_col, bm, tile8, expi64, bm8_64, e8v), 0.0)
    out3 = _gat(x2, gs[2], mask_r, ea_r, loop_col, bm, tile8, expi3, bm8_3, e8v)
    rec = out3 + 0.1 * (_dot(latent, skip_W[...]) + skip_b[...])
    rec_ref[...] = rec


_kk = np.arange(_N)
_ipn = (np.arange(_P) // _N)[None, :]
_jpn = (np.arange(_P) % _N)[None, :]
_k1 = _kk[:, None]
_NOTEXCL = np.where((_k1 == _ipn) | (_k1 == _jpn), 0.0, 1.0).astype(np.float32)[None]
_SAVG = (0.5 * ((_k1 == _ipn) + (_k1 == _jpn))).astype(np.float32)
_DMAT = ((_k1 == _ipn).astype(np.float32) - (_k1 == _jpn).astype(np.float32))
_r = np.arange(_RB)
_BM = ((_r[:, None] // _N) == (_r[None, :] // _N)).astype(np.float32)
_TILE8 = (_kk[:, None] == (_r[None, :] % _N)).astype(np.float32)
_EXPI64 = (_kk[:, None] == (np.arange(_N * 64) // 64)[None, :]).astype(np.float32)
_BM8_64 = ((np.arange(_N * 64) // 64)[:, None] == _kk[None, :]).astype(np.float32)
_EXPI3 = (_kk[:, None] == (np.arange(_N * 3) // 3)[None, :]).astype(np.float32)
_BM8_3 = ((np.arange(_N * 3) // 3)[:, None] == _kk[None, :]).astype(np.float32)
_E8V = ((_r[:, None] // _N) == np.arange(_B)[None, :]).astype(np.float32)
_PREFIX = np.stack(
    [np.zeros(_RB, np.float32), np.tile(np.arange(_N, dtype=np.float32), _B)],
    axis=1)


def _full(shape):
    nd = len(shape)
    return pl.BlockSpec(shape, lambda i, _nd=nd: (0,) * _nd)


def kernel(x, params):
    B, N = x.shape
    p = params
    plist = [p['enc_W1'], p['enc_b1'].reshape(1, -1),
             p['enc_W2'], p['enc_b2'].reshape(1, -1),
             p['enc_W3'], p['enc_b3'].reshape(1, -1),
             p['skip_W'], p['skip_b'].reshape(1, -1)]
    for gk in ('g1', 'g2', 'g3'):
        g = p[gk]
        plist += [g['lin_l_W'], g['lin_l_b'].reshape(1, -1),
                  g['lin_r_W'], g['lin_r_b'].reshape(1, -1),
                  g['lin_edge_W'],
                  g['att'].reshape(1, -1), g['att'].reshape(-1, 1),
                  g['bias'].reshape(1, -1)]

    row_spec = pl.BlockSpec((_RB, 3), lambda i: (i, 0))
    const_in = [jnp.asarray(c) for c in
                (_PREFIX, _NOTEXCL, _BM, _SAVG, _DMAT, _TILE8,
                 _EXPI64, _BM8_64, _EXPI3, _BM8_3, _E8V)]
    xp_flat, rec_flat, lat_flat = pl.pallas_call(
        _body,
        grid=(1,),
        in_specs=([pl.BlockSpec((_RB, 1), lambda i: (i, 0))]
                  + [_full(c.shape) for c in const_in]
                  + [_full(l.shape) for l in plist]),
        out_specs=[row_spec, row_spec, row_spec],
        out_shape=[
            jax.ShapeDtypeStruct((_RB, 3), jnp.float32),
            jax.ShapeDtypeStruct((_RB, 3), jnp.float32),
            jax.ShapeDtypeStruct((_RB, 3), jnp.float32),
        ],
    )(x.reshape(_RB, 1), *const_in, *plist)

    return (xp_flat.reshape(B, N, 3), rec_flat.reshape(B, N, 3),
            lat_flat.reshape(B, N, 3))
